# Initial kernel scaffold; baseline (speedup 1.0000x reference)
import functools

import jax
import jax.numpy as jnp
from jax import lax
from jax.experimental import pallas as pl
from jax.experimental.pallas import tpu as pltpu
from jax.experimental.pallas import tpu_sc as plsc

VOCAB = 1000000
DIM = 32
BATCH = 4096
HIST = 200

NUM_CORES = 2
NUM_SUBCORES = 16
NW = NUM_CORES * NUM_SUBCORES

B_TOTAL = BATCH * HIST
B_PER_W = B_TOTAL // NW
CHUNK = 1280
N_CHUNKS = B_PER_W // CHUNK


def _make_gather():
  mesh = plsc.VectorSubcoreMesh(core_axis_name="c", subcore_axis_name="s")

  @functools.partial(
      pl.kernel,
      mesh=mesh,
      out_type=jax.ShapeDtypeStruct((B_TOTAL, DIM), jnp.float32),
      scratch_types=[
          pltpu.VMEM((B_PER_W,), jnp.int32),
          pltpu.VMEM((CHUNK, DIM), jnp.float32),
          pltpu.SemaphoreType.DMA,
      ],
  )
  def gather_kernel(idx_hbm, table_hbm, out_hbm, idx_v, rows_v, sem):
    wid = lax.axis_index("s") * NUM_CORES + lax.axis_index("c")
    base = wid * B_PER_W
    pltpu.sync_copy(idx_hbm.at[pl.ds(base, B_PER_W)], idx_v)

    def chunk_body(g, carry):
      off = g * CHUNK
      pltpu.async_copy(
          table_hbm.at[idx_v.at[pl.ds(off, CHUNK)]], rows_v, sem
      ).wait()
      pltpu.sync_copy(rows_v, out_hbm.at[pl.ds(base + off, CHUNK)])
      return carry

    lax.fori_loop(0, N_CHUNKS, chunk_body, 0)

  return gather_kernel


_gather = _make_gather()


@jax.jit
def kernel(inputs, table):
  flat_idx = inputs.reshape(B_TOTAL).astype(jnp.int32)
  out = _gather(flat_idx, table)
  return out.reshape(BATCH, HIST, DIM)


# speedup vs baseline: 1.4820x; 1.4820x over previous
import functools

import jax
import jax.numpy as jnp
from jax import lax
from jax.experimental import pallas as pl
from jax.experimental.pallas import tpu as pltpu
from jax.experimental.pallas import tpu_sc as plsc

VOCAB = 1000000
DIM = 32
BATCH = 4096
HIST = 200

NUM_CORES = 2
NUM_SUBCORES = 16
NW = NUM_CORES * NUM_SUBCORES

B_TOTAL = BATCH * HIST
B_PER_W = B_TOTAL // NW
CHUNK = 1280
N_CHUNKS = B_PER_W // CHUNK


def _make_gather():
  mesh = plsc.VectorSubcoreMesh(core_axis_name="c", subcore_axis_name="s")

  @functools.partial(
      pl.kernel,
      mesh=mesh,
      out_type=jax.ShapeDtypeStruct((B_TOTAL, DIM), jnp.float32),
      scratch_types=[
          pltpu.VMEM((B_PER_W,), jnp.int32),
          pltpu.VMEM((CHUNK, DIM), jnp.float32),
          pltpu.SemaphoreType.DMA,
      ],
      compiler_params=pltpu.CompilerParams(use_tc_tiling_on_sc=False),
  )
  def gather_kernel(idx_hbm, table_hbm, out_hbm, idx_v, rows_v, sem):
    wid = lax.axis_index("s") * NUM_CORES + lax.axis_index("c")
    base = wid * B_PER_W
    pltpu.sync_copy(idx_hbm.at[pl.ds(base, B_PER_W)], idx_v)

    def chunk_body(g, carry):
      off = g * CHUNK
      pltpu.async_copy(
          table_hbm.at[idx_v.at[pl.ds(off, CHUNK)]], rows_v, sem
      ).wait()
      pltpu.sync_copy(rows_v, out_hbm.at[pl.ds(base + off, CHUNK)])
      return carry

    lax.fori_loop(0, N_CHUNKS, chunk_body, 0)

  return gather_kernel


_gather = _make_gather()


@jax.jit
def kernel(inputs, table):
  flat_idx = inputs.reshape(B_TOTAL).astype(jnp.int32)
  out = _gather(flat_idx, table)
  return out.reshape(BATCH, HIST, DIM)


# speedup vs baseline: 1.4988x; 1.0113x over previous
import functools

import jax
import jax.numpy as jnp
from jax import lax
from jax.experimental import pallas as pl
from jax.experimental.pallas import tpu as pltpu
from jax.experimental.pallas import tpu_sc as plsc

VOCAB = 1000000
DIM = 32
BATCH = 4096
HIST = 200

NUM_CORES = 2
NUM_SUBCORES = 16
NW = NUM_CORES * NUM_SUBCORES

B_TOTAL = BATCH * HIST
B_PER_W = B_TOTAL // NW
CHUNK = 640
N_CHUNKS = B_PER_W // CHUNK
NBUF = 4
PRIME = 2


def _make_gather():
  mesh = plsc.VectorSubcoreMesh(core_axis_name="c", subcore_axis_name="s")

  @functools.partial(
      pl.kernel,
      mesh=mesh,
      out_type=jax.ShapeDtypeStruct((B_TOTAL, DIM), jnp.float32),
      scratch_types=[
          pltpu.VMEM((B_PER_W,), jnp.int32),
          pltpu.VMEM((NBUF, CHUNK, DIM), jnp.float32),
      ]
      + [pltpu.SemaphoreType.DMA] * (2 * NBUF),
      compiler_params=pltpu.CompilerParams(use_tc_tiling_on_sc=False),
  )
  def gather_kernel(idx_hbm, table_hbm, out_hbm, idx_v, rows_v, *sems):
    gs, ws = sems[:NBUF], sems[NBUF:]
    wid = lax.axis_index("s") * NUM_CORES + lax.axis_index("c")
    base = wid * B_PER_W
    pltpu.sync_copy(idx_hbm.at[pl.ds(base, B_PER_W)], idx_v)

    gcp = [None] * NBUF
    wcp = [None] * NBUF

    def gather_start(g):
      b = g % NBUF
      gcp[b] = pltpu.async_copy(
          table_hbm.at[idx_v.at[pl.ds(g * CHUNK, CHUNK)]], rows_v.at[b], gs[b]
      )

    def write_start(g):
      b = g % NBUF
      wcp[b] = pltpu.async_copy(
          rows_v.at[b], out_hbm.at[pl.ds(base + g * CHUNK, CHUNK)], ws[b]
      )

    for g in range(min(PRIME, N_CHUNKS)):
      gather_start(g)
    for g in range(N_CHUNKS):
      nxt = g + PRIME
      if nxt < N_CHUNKS:
        if nxt >= NBUF:
          wcp[nxt % NBUF].wait()
        gather_start(nxt)
      gcp[g % NBUF].wait()
      write_start(g)
    for g in range(max(0, N_CHUNKS - NBUF), N_CHUNKS):
      wcp[g % NBUF].wait()

  return gather_kernel


_gather = _make_gather()


@jax.jit
def kernel(inputs, table):
  flat_idx = inputs.reshape(B_TOTAL).astype(jnp.int32)
  out = _gather(flat_idx, table)
  return out.reshape(BATCH, HIST, DIM)


# speedup vs baseline: 1.5006x; 1.0013x over previous
import functools

import jax
import jax.numpy as jnp
from jax import lax
from jax.experimental import pallas as pl
from jax.experimental.pallas import tpu as pltpu
from jax.experimental.pallas import tpu_sc as plsc

VOCAB = 1000000
DIM = 32
BATCH = 4096
HIST = 200

NUM_CORES = 2
NUM_SUBCORES = 16
NW = NUM_CORES * NUM_SUBCORES

B_TOTAL = BATCH * HIST
B_PER_W = B_TOTAL // NW
CHUNK = 320
N_CHUNKS = B_PER_W // CHUNK
NBUF = 8
PRIME = 6


def _make_gather():
  mesh = plsc.VectorSubcoreMesh(core_axis_name="c", subcore_axis_name="s")

  @functools.partial(
      pl.kernel,
      mesh=mesh,
      out_type=jax.ShapeDtypeStruct((B_TOTAL, DIM), jnp.float32),
      scratch_types=[
          pltpu.VMEM((B_PER_W,), jnp.int32),
          pltpu.VMEM((NBUF, CHUNK, DIM), jnp.float32),
      ]
      + [pltpu.SemaphoreType.DMA] * (2 * NBUF),
      compiler_params=pltpu.CompilerParams(use_tc_tiling_on_sc=False),
  )
  def gather_kernel(idx_hbm, table_hbm, out_hbm, idx_v, rows_v, *sems):
    gs, ws = sems[:NBUF], sems[NBUF:]
    wid = lax.axis_index("s") * NUM_CORES + lax.axis_index("c")
    base = wid * B_PER_W
    pltpu.sync_copy(idx_hbm.at[pl.ds(base, B_PER_W)], idx_v)

    gcp = [None] * NBUF
    wcp = [None] * NBUF

    def gather_start(g):
      b = g % NBUF
      gcp[b] = pltpu.async_copy(
          table_hbm.at[idx_v.at[pl.ds(g * CHUNK, CHUNK)]], rows_v.at[b], gs[b]
      )

    def write_start(g):
      b = g % NBUF
      wcp[b] = pltpu.async_copy(
          rows_v.at[b], out_hbm.at[pl.ds(base + g * CHUNK, CHUNK)], ws[b]
      )

    for g in range(min(PRIME, N_CHUNKS)):
      gather_start(g)
    for g in range(N_CHUNKS):
      nxt = g + PRIME
      if nxt < N_CHUNKS:
        if nxt >= NBUF:
          wcp[nxt % NBUF].wait()
        gather_start(nxt)
      gcp[g % NBUF].wait()
      write_start(g)
    for g in range(max(0, N_CHUNKS - NBUF), N_CHUNKS):
      wcp[g % NBUF].wait()

  return gather_kernel


_gather = _make_gather()


@jax.jit
def kernel(inputs, table):
  flat_idx = inputs.reshape(B_TOTAL).astype(jnp.int32)
  out = _gather(flat_idx, table)
  return out.reshape(BATCH, HIST, DIM)
